# baseline (speedup 1.0000x reference)
import functools

import jax
import jax.numpy as jnp
import numpy as np
from jax import lax
from jax.experimental import pallas as pl
from jax.experimental.pallas import tpu as pltpu
from jax.experimental.pallas import tpu_sc as plsc

_BN = 1.0 / np.sqrt(1.0 + 1e-5)
_N = 10000
_E = 320000
_EH = _E // 2
_NC = 2
_NS = 16
_NW = _NC * _NS
_EPW = _EH // _NW
_CHUNK = 40
_ITERS = _EPW // _CHUNK

_mesh = plsc.VectorSubcoreMesh(core_axis_name="c", subcore_axis_name="s")


def _enc_body(x_ref, w1_ref, b1_ref, w2_ref, b2_ref, wtr_ref, wtc_ref,
              xe_ref, tr_ref, tc_ref):
    h = jnp.maximum((x_ref[...] @ w1_ref[...] + b1_ref[...]) * _BN, 0.0)
    xe = h @ w2_ref[...] + b2_ref[...]
    xe_ref[...] = xe
    tr_ref[...] = xe @ wtr_ref[...]
    tc_ref[...] = xe @ wtc_ref[...]


def _encode(x, w1, b1, w2, b2, wtr, wtc):
    nb = 1000
    grid = _N // nb
    full = lambda a: pl.BlockSpec(a.shape, lambda i: (0,) * a.ndim)
    return pl.pallas_call(
        _enc_body,
        grid=(grid,),
        in_specs=[pl.BlockSpec((nb, 128), lambda i: (i, 0)),
                  full(w1), full(b1), full(w2), full(b2), full(wtr), full(wtc)],
        out_specs=[pl.BlockSpec((nb, 64), lambda i: (i, 0)),
                   pl.BlockSpec((nb, 128), lambda i: (i, 0)),
                   pl.BlockSpec((nb, 128), lambda i: (i, 0))],
        out_shape=[jax.ShapeDtypeStruct((_N, 64), jnp.float32),
                   jax.ShapeDtypeStruct((_N, 128), jnp.float32),
                   jax.ShapeDtypeStruct((_N, 128), jnp.float32)],
    )(x, w1, b1, w2, b2, wtr, wtc)


@functools.partial(
    pl.kernel,
    out_type=jax.ShapeDtypeStruct((_EH, 128), jnp.float32),
    mesh=_mesh,
    scratch_types=[
        pltpu.VMEM((2 * _EPW,), jnp.int32),
        pltpu.VMEM((_CHUNK, 128), jnp.float32),
        pltpu.VMEM((_CHUNK, 128), jnp.float32),
        pltpu.VMEM((_CHUNK, 128), jnp.float32),
        pltpu.VMEM((_CHUNK, 128), jnp.float32),
        pltpu.VMEM((_CHUNK, 128), jnp.float32),
        pltpu.VMEM((_CHUNK, 128), jnp.float32),
        pltpu.SemaphoreType.DMA,
        pltpu.SemaphoreType.DMA,
        pltpu.SemaphoreType.DMA,
        pltpu.SemaphoreType.DMA,
        pltpu.SemaphoreType.DMA,
        pltpu.SemaphoreType.DMA,
        pltpu.SemaphoreType.DMA,
        pltpu.SemaphoreType.DMA,
        pltpu.SemaphoreType.DMA,
    ],
)
def _sc_gather(tr_hbm, tcp_hbm, idx_hbm, g_out, idx_all,
               br0, br1, br2, bc0, bc1, bc2,
               sr0, sr1, sr2, sc0, sc1, sc2, sw0, sw1, sw2):
    wid = lax.axis_index("s") * _NC + lax.axis_index("c")
    base0 = wid * _EPW
    brs = (br0, br1, br2)
    bcs = (bc0, bc1, bc2)
    srs = (sr0, sr1, sr2)
    scs = (sc0, sc1, sc2)
    sws = (sw0, sw1, sw2)

    pltpu.sync_copy(idx_hbm.at[pl.ds(wid * 2 * _EPW, 2 * _EPW)], idx_all)

    def _idr(i):
        return idx_all.at[pl.ds(i * 2 * _CHUNK, _CHUNK)]

    def _idc(i):
        return idx_all.at[pl.ds(i * 2 * _CHUNK + _CHUNK, _CHUNK)]

    def start(i, s):
        pltpu.async_copy(tr_hbm.at[_idr(i)], brs[s], srs[s])
        pltpu.async_copy(tcp_hbm.at[_idc(i)], bcs[s], scs[s])

    def finish(i, s):
        pltpu.make_async_copy(tr_hbm.at[_idr(i)], brs[s], srs[s]).wait()
        pltpu.make_async_copy(tcp_hbm.at[_idc(i)], bcs[s], scs[s]).wait()
        br = brs[s]
        bc = bcs[s]

        def addbody(e4, c2):
            for e1 in range(4):
                for c4 in range(4):
                    sl = pl.ds(c4 * 16, 16)
                    e = e4 * 4 + e1
                    br[e, sl] = br[e, sl] + bc[e, sl]
            return c2

        lax.fori_loop(0, _CHUNK // 4, addbody, 0)
        pltpu.async_copy(br, g_out.at[pl.ds(base0 + i * _CHUNK, _CHUNK)],
                         sws[s])

    def wait_wb(i, s):
        pltpu.make_async_copy(
            brs[s], g_out.at[pl.ds(base0 + i * _CHUNK, _CHUNK)],
            sws[s]).wait()

    start(0, 0)

    def body(j, carry):
        i = 3 * j

        @pl.when(j > 0)
        def _():
            wait_wb(i - 2, 1)

        start(i + 1, 1)
        finish(i, 0)

        @pl.when(j > 0)
        def _():
            wait_wb(i - 1, 2)

        start(i + 2, 2)
        finish(i + 1, 1)

        wait_wb(i, 0)
        start(i + 3, 0)
        finish(i + 2, 2)
        return carry

    lax.fori_loop(0, (_ITERS - 2) // 3, body, 0)
    last = _ITERS - 2
    wait_wb(last - 2, 1)
    start(last + 1, 1)
    finish(last, 0)
    finish(last + 1, 1)
    wait_wb(last - 1, 2)
    wait_wb(last, 0)
    wait_wb(last + 1, 1)


def _edge_body(gr_ref, ea_ref, ew1_ref, eb1_ref, we_ref, cb1_ref,
               emw2_ref, emb2_ref, be_ref, nb1_ref, nw2_ref, nb2_ref,
               ea2_ref, h_ref, easum_ref):
    eh = jnp.maximum((ea_ref[...] @ ew1_ref[...] + eb1_ref[...]) * _BN, 0.0)
    gr = gr_ref[...]
    hem = jnp.maximum(
        (gr[:, :64] + eh @ we_ref[...] + cb1_ref[...]) * _BN, 0.0)
    ea2 = hem @ emw2_ref[...] + emb2_ref[...]
    hnm = jnp.maximum(
        (gr[:, 64:] + ea2 @ be_ref[...] + nb1_ref[...]) * _BN, 0.0)
    ea2_ref[...] = ea2
    h = hnm @ nw2_ref[...] + nb2_ref[...]
    h_ref[...] = jnp.concatenate([h, jnp.zeros_like(h)], axis=1)

    @pl.when(pl.program_id(0) == 0)
    def _():
        easum_ref[...] = jnp.zeros_like(easum_ref)

    easum_ref[...] += jnp.sum(ea2, axis=0, keepdims=True)


def _edge_stage(gr, ea, ew1, eb1, we, cb1, emw2, emb2, be, nb1, nw2, nb2):
    eb = 2000
    grid = _EH // eb
    full = lambda a: pl.BlockSpec(a.shape, lambda i: (0,) * a.ndim)
    return pl.pallas_call(
        _edge_body,
        grid=(grid,),
        in_specs=[pl.BlockSpec((eb, 128), lambda i: (i, 0)),
                  pl.BlockSpec((eb, 16), lambda i: (i, 0)),
                  full(ew1), full(eb1), full(we), full(cb1), full(emw2),
                  full(emb2), full(be), full(nb1), full(nw2), full(nb2)],
        out_specs=[pl.BlockSpec((eb, 16), lambda i: (i, 0)),
                   pl.BlockSpec((eb, 128), lambda i: (i, 0)),
                   pl.BlockSpec((1, 16), lambda i: (0, 0))],
        out_shape=[jax.ShapeDtypeStruct((_EH, 16), jnp.float32),
                   jax.ShapeDtypeStruct((_EH, 128), jnp.float32),
                   jax.ShapeDtypeStruct((1, 16), jnp.float32)],
    )(gr, ea, ew1, eb1, we, cb1, emw2, emb2, be, nb1, nw2, nb2)


@functools.partial(
    pl.kernel,
    out_type=jax.ShapeDtypeStruct((2, _N, 128), jnp.float32),
    mesh=_mesh,
    scratch_types=[
        pltpu.VMEM((_ITERS, _CHUNK), jnp.int32),
        pltpu.VMEM((_CHUNK, 128), jnp.float32),
        pltpu.VMEM((_CHUNK, 128), jnp.float32),
        pltpu.VMEM((_CHUNK, 128), jnp.float32),
        pltpu.VMEM_SHARED((_N, 128), jnp.float32),
        pltpu.SemaphoreType.DMA,
        pltpu.SemaphoreType.DMA,
        pltpu.SemaphoreType.DMA,
        pltpu.SemaphoreType.DMA,
        pltpu.SemaphoreType.DMA,
        pltpu.SemaphoreType.DMA,
    ],
)
def _sc_scatter(h_hbm, col3_hbm, zeros_hbm, out_hbm, colv, hv0, hv1, hv2,
                acc, sh0, sh1, sh2, ss0, ss1, ss2):
    cid = lax.axis_index("c")
    sid = lax.axis_index("s")
    wid = sid * _NC + cid
    hvs = (hv0, hv1, hv2)
    shs = (sh0, sh1, sh2)
    sss = (ss0, ss1, ss2)

    @pl.when(sid == 0)
    def _():
        pltpu.sync_copy(zeros_hbm, acc)

    pltpu.sync_copy(col3_hbm.at[wid], colv)
    plsc.subcore_barrier()

    base0 = wid * _EPW

    def start(i, s):
        pltpu.async_copy(h_hbm.at[pl.ds(base0 + i * _CHUNK, _CHUNK)],
                         hvs[s], shs[s])

    def finish(i, s):
        pltpu.make_async_copy(
            h_hbm.at[pl.ds(base0 + i * _CHUNK, _CHUNK)], hvs[s],
            shs[s]).wait()
        pltpu.async_copy(hvs[s], acc.at[colv.at[i]], sss[s], add=True)

    def wait_sc(i, s):
        pltpu.make_async_copy(hvs[s], acc.at[colv.at[i]], sss[s]).wait()

    start(0, 0)

    def body(j, carry):
        i = 3 * j

        @pl.when(j > 0)
        def _():
            wait_sc(i - 2, 1)

        start(i + 1, 1)
        finish(i, 0)

        @pl.when(j > 0)
        def _():
            wait_sc(i - 1, 2)

        start(i + 2, 2)
        finish(i + 1, 1)

        wait_sc(i, 0)
        start(i + 3, 0)
        finish(i + 2, 2)
        return carry

    lax.fori_loop(0, (_ITERS - 2) // 3, body, 0)
    last = _ITERS - 2
    wait_sc(last - 2, 1)
    start(last + 1, 1)
    finish(last, 0)
    finish(last + 1, 1)
    wait_sc(last - 1, 2)
    wait_sc(last, 0)
    wait_sc(last + 1, 1)
    plsc.subcore_barrier()

    @pl.when(sid == 0)
    def _():
        pltpu.sync_copy(acc, out_hbm.at[cid])


def _final_body(xe_ref, pa_ref, pb_ref, easum_ref, cx_ref, ca_ref, nb1_ref,
                nw2_ref, nb2_ref, gx_ref, ge_ref, gb1_ref, gw2_ref, gb2_ref,
                sw1_ref, sb1_ref, sw2_ref, sb2_ref,
                xo_ref, sh_ref, u_ref, nacc_ref):
    agg = (pa_ref[0][:, :64] + pa_ref[1][:, :64]
           + pb_ref[0][:, :64] + pb_ref[1][:, :64])
    h2 = jnp.maximum(
        (xe_ref[...] @ cx_ref[...] + agg @ ca_ref[...] + nb1_ref[...]) * _BN,
        0.0)
    xo = h2 @ nw2_ref[...] + nb2_ref[...]
    xo_ref[...] = xo
    hs = jnp.maximum(xo @ sw1_ref[...] + sb1_ref[...], 0.0)
    sh_ref[...] = hs @ sw2_ref[...] + sb2_ref[...]

    i = pl.program_id(0)

    @pl.when(i == 0)
    def _():
        nacc_ref[...] = jnp.zeros_like(nacc_ref)

    nacc_ref[...] += jnp.sum(xo, axis=0, keepdims=True)

    @pl.when(i == pl.num_programs(0) - 1)
    def _():
        node_mean = nacc_ref[...] * (1.0 / _N)
        edge_mean = easum_ref[...] * (1.0 / _E)
        gh = jnp.maximum(
            (node_mean @ gx_ref[...] + edge_mean @ ge_ref[...] + gb1_ref[...])
            * _BN, 0.0)
        u_ref[...] = gh @ gw2_ref[...] + gb2_ref[...]


def _final_stage(xe, pa, pb, easum, cx, ca, nb1, nw2, nb2, gx, ge, gb1, gw2,
                 gb2, sw1, sb1, sw2, sb2):
    nb = 1000
    grid = _N // nb
    full = lambda a: pl.BlockSpec(a.shape, lambda i: (0,) * a.ndim)
    return pl.pallas_call(
        _final_body,
        grid=(grid,),
        in_specs=[pl.BlockSpec((nb, 64), lambda i: (i, 0)),
                  pl.BlockSpec((2, nb, 128), lambda i: (0, i, 0)),
                  pl.BlockSpec((2, nb, 128), lambda i: (0, i, 0)),
                  full(easum), full(cx), full(ca), full(nb1), full(nw2),
                  full(nb2), full(gx), full(ge), full(gb1), full(gw2),
                  full(gb2), full(sw1), full(sb1), full(sw2), full(sb2)],
        out_specs=[pl.BlockSpec((nb, 64), lambda i: (i, 0)),
                   pl.BlockSpec((nb, 1), lambda i: (i, 0)),
                   pl.BlockSpec((1, 64), lambda i: (0, 0))],
        out_shape=[jax.ShapeDtypeStruct((_N, 64), jnp.float32),
                   jax.ShapeDtypeStruct((_N, 1), jnp.float32),
                   jax.ShapeDtypeStruct((1, 64), jnp.float32)],
        scratch_shapes=[pltpu.VMEM((1, 64), jnp.float32)],
    )(xe, pa, pb, easum, cx, ca, nb1, nw2, nb2, gx, ge, gb1, gw2, gb2, sw1,
      sb1, sw2, sb2)


def _idx_stream(r, c):
    return jnp.stack([r.reshape(-1, _CHUNK), c.reshape(-1, _CHUNK)],
                     axis=1).reshape(-1)


def kernel(x, edge_index, edge_attr, params):
    p = params
    row = edge_index[0]
    col = edge_index[1]
    r1 = lambda b: b.reshape(1, -1)

    a_r = p['em_W1'][:64]
    a_c = p['em_W1'][64:128]
    a_e = p['em_W1'][128:]
    b_r = p['nm1_W1'][:64]
    b_e = p['nm1_W1'][64:]
    wtr = jnp.concatenate([a_r, b_r], axis=1)
    w_e = p['ee_W2'] @ a_e
    cb1 = r1(p['ee_b2'] @ a_e + p['em_b1'])
    wtc = jnp.concatenate([a_c, jnp.zeros((64, 64), jnp.float32)], axis=1)

    idx_a = _idx_stream(row[:_EH], col[:_EH])
    idx_b = _idx_stream(row[_EH:], col[_EH:])
    col3_a = col[:_EH].reshape(_NW, _ITERS, _CHUNK)
    col3_b = col[_EH:].reshape(_NW, _ITERS, _CHUNK)
    zeros = jnp.zeros((_N, 128), jnp.float32)

    xe, tr, tcp = _encode(x, p['ne_W1'], r1(p['ne_b1']), p['ne_W2'],
                          r1(p['ne_b2']), wtr, wtc)
    g_a = _sc_gather(tr, tcp, idx_a)
    g_b = _sc_gather(tr, tcp, idx_b)
    edge_args = (p['ee_W1'], r1(p['ee_b1']), w_e, cb1,
                 p['em_W2'], r1(p['em_b2']), b_e, r1(p['nm1_b1']),
                 p['nm1_W2'], r1(p['nm1_b2']))
    ea2_a, h_a, easum_a = _edge_stage(g_a, edge_attr[:_EH], *edge_args)
    ea2_b, h_b, easum_b = _edge_stage(g_b, edge_attr[_EH:], *edge_args)
    parts_a = _sc_scatter(h_a, col3_a, zeros)
    parts_b = _sc_scatter(h_b, col3_b, zeros)
    ea2 = jnp.concatenate([ea2_a, ea2_b], axis=0)
    easum = easum_a + easum_b
    xo, shifts, u = _final_stage(
        xe, parts_a, parts_b, easum, p['nm2_W1'][:64], p['nm2_W1'][64:],
        r1(p['nm2_b1']), p['nm2_W2'], r1(p['nm2_b2']),
        p['gm_W1'][:64], p['gm_W1'][64:], r1(p['gm_b1']), p['gm_W2'],
        r1(p['gm_b2']), p['sp_W1'], r1(p['sp_b1']), p['sp_W2'],
        r1(p['sp_b2']))
    return (shifts, (xo, ea2, u))


# speedup vs baseline: 1.3818x; 1.3818x over previous
import functools

import jax
import jax.numpy as jnp
import numpy as np
from jax import lax
from jax.experimental import pallas as pl
from jax.experimental.pallas import tpu as pltpu
from jax.experimental.pallas import tpu_sc as plsc

_BN = 1.0 / np.sqrt(1.0 + 1e-5)
_N = 10000
_E = 320000
_EH = _E // 2
_NC = 2
_NS = 16
_NW = _NC * _NS
_EPW = _EH // _NW
_CHUNK = 40
_ITERS = _EPW // _CHUNK

_mesh = plsc.VectorSubcoreMesh(core_axis_name="c", subcore_axis_name="s")


def _enc_body(x_ref, w1_ref, b1_ref, w2_ref, b2_ref, wtr_ref, wtc_ref,
              xe_ref, tr_ref, tc_ref):
    h = jnp.maximum((x_ref[...] @ w1_ref[...] + b1_ref[...]) * _BN, 0.0)
    xe = h @ w2_ref[...] + b2_ref[...]
    xe_ref[...] = xe
    tr_ref[...] = xe @ wtr_ref[...]
    tc_ref[...] = xe @ wtc_ref[...]


def _encode(x, w1, b1, w2, b2, wtr, wtc):
    nb = 1000
    grid = _N // nb
    full = lambda a: pl.BlockSpec(a.shape, lambda i: (0,) * a.ndim)
    return pl.pallas_call(
        _enc_body,
        grid=(grid,),
        in_specs=[pl.BlockSpec((nb, 128), lambda i: (i, 0)),
                  full(w1), full(b1), full(w2), full(b2), full(wtr), full(wtc)],
        out_specs=[pl.BlockSpec((nb, 64), lambda i: (i, 0)),
                   pl.BlockSpec((nb, 128), lambda i: (i, 0)),
                   pl.BlockSpec((nb, 128), lambda i: (i, 0))],
        out_shape=[jax.ShapeDtypeStruct((_N, 64), jnp.float32),
                   jax.ShapeDtypeStruct((_N, 128), jnp.float32),
                   jax.ShapeDtypeStruct((_N, 128), jnp.float32)],
    )(x, w1, b1, w2, b2, wtr, wtc)


@functools.partial(
    pl.kernel,
    out_type=jax.ShapeDtypeStruct((_EH, 128), jnp.float32),
    mesh=_mesh,
    scratch_types=[
        pltpu.VMEM((2 * _EPW,), jnp.int32),
        pltpu.VMEM((_CHUNK, 128), jnp.float32),
        pltpu.VMEM((_CHUNK, 128), jnp.float32),
        pltpu.VMEM((_CHUNK, 128), jnp.float32),
        pltpu.VMEM((_CHUNK, 128), jnp.float32),
        pltpu.VMEM((_CHUNK, 128), jnp.float32),
        pltpu.VMEM((_CHUNK, 128), jnp.float32),
        pltpu.SemaphoreType.DMA,
        pltpu.SemaphoreType.DMA,
        pltpu.SemaphoreType.DMA,
        pltpu.SemaphoreType.DMA,
        pltpu.SemaphoreType.DMA,
        pltpu.SemaphoreType.DMA,
        pltpu.SemaphoreType.DMA,
        pltpu.SemaphoreType.DMA,
        pltpu.SemaphoreType.DMA,
    ],
)
def _sc_gather(tr_hbm, tcp_hbm, idx_hbm, g_out, idx_all,
               br0, br1, br2, bc0, bc1, bc2,
               sr0, sr1, sr2, sc0, sc1, sc2, sw0, sw1, sw2):
    wid = lax.axis_index("s") * _NC + lax.axis_index("c")
    base0 = wid * _EPW
    brs = (br0, br1, br2)
    bcs = (bc0, bc1, bc2)
    srs = (sr0, sr1, sr2)
    scs = (sc0, sc1, sc2)
    sws = (sw0, sw1, sw2)

    pltpu.sync_copy(idx_hbm.at[pl.ds(wid * 2 * _EPW, 2 * _EPW)], idx_all)

    def _idr(i):
        return idx_all.at[pl.ds(i * 2 * _CHUNK, _CHUNK)]

    def _idc(i):
        return idx_all.at[pl.ds(i * 2 * _CHUNK + _CHUNK, _CHUNK)]

    def start(i, s):
        pltpu.async_copy(tr_hbm.at[_idr(i)], brs[s], srs[s])
        pltpu.async_copy(tcp_hbm.at[_idc(i)], bcs[s], scs[s])

    def finish(i, s):
        pltpu.make_async_copy(tr_hbm.at[_idr(i)], brs[s], srs[s]).wait()
        pltpu.make_async_copy(tcp_hbm.at[_idc(i)], bcs[s], scs[s]).wait()
        br = brs[s]
        bc = bcs[s]

        def addbody(e4, c2):
            for e1 in range(4):
                for c4 in range(4):
                    sl = pl.ds(c4 * 16, 16)
                    e = e4 * 4 + e1
                    br[e, sl] = br[e, sl] + bc[e, sl]
            return c2

        lax.fori_loop(0, _CHUNK // 4, addbody, 0)
        pltpu.async_copy(br, g_out.at[pl.ds(base0 + i * _CHUNK, _CHUNK)],
                         sws[s])

    def wait_wb(i, s):
        pltpu.make_async_copy(
            brs[s], g_out.at[pl.ds(base0 + i * _CHUNK, _CHUNK)],
            sws[s]).wait()

    start(0, 0)

    def body(j, carry):
        i = 3 * j

        @pl.when(j > 0)
        def _():
            wait_wb(i - 2, 1)

        start(i + 1, 1)
        finish(i, 0)

        @pl.when(j > 0)
        def _():
            wait_wb(i - 1, 2)

        start(i + 2, 2)
        finish(i + 1, 1)

        wait_wb(i, 0)
        start(i + 3, 0)
        finish(i + 2, 2)
        return carry

    lax.fori_loop(0, (_ITERS - 2) // 3, body, 0)
    last = _ITERS - 2
    wait_wb(last - 2, 1)
    start(last + 1, 1)
    finish(last, 0)
    finish(last + 1, 1)
    wait_wb(last - 1, 2)
    wait_wb(last, 0)
    wait_wb(last + 1, 1)


def _edge_body(gr_ref, ea_ref, ew1_ref, eb1_ref, we_ref, cb1_ref,
               emw2_ref, emb2_ref, be_ref, nb1_ref, nw2_ref, nb2_ref,
               ea2_ref, h_ref, easum_ref):
    dg = lax.dot_general
    c00 = (((0,), (0,)), ((), ()))
    eht = jnp.maximum(
        (dg(ew1_ref[...], ea_ref[...], c00) + eb1_ref[...]) * _BN, 0.0)
    gr = gr_ref[...]
    contrib = dg(eht, we_ref[...], c00)
    hem = jnp.maximum((gr[:, :64] + contrib + cb1_ref[...]) * _BN, 0.0)
    ea2t = dg(emw2_ref[...], hem, (((0,), (1,)), ((), ()))) + emb2_ref[...]
    ea2_ref[...] = ea2t
    t2 = dg(ea2t, be_ref[...], c00)
    hnm = jnp.maximum((gr[:, 64:] + t2 + nb1_ref[...]) * _BN, 0.0)
    h = hnm @ nw2_ref[...] + nb2_ref[...]
    h_ref[...] = jnp.concatenate([h, jnp.zeros_like(h)], axis=1)

    @pl.when(pl.program_id(0) == 0)
    def _():
        easum_ref[...] = jnp.zeros_like(easum_ref)

    easum_ref[...] += jnp.sum(ea2t, axis=1, keepdims=True)


def _edge_stage(gr, ea_t, half, ew1, eb1, we, cb1, emw2, emb2, be, nb1, nw2,
                nb2):
    eb = 3200
    grid = _EH // eb
    off = half * grid
    full = lambda a: pl.BlockSpec(a.shape, lambda i: (0,) * a.ndim)
    return pl.pallas_call(
        _edge_body,
        grid=(grid,),
        in_specs=[pl.BlockSpec((eb, 128), lambda i: (i, 0)),
                  pl.BlockSpec((16, eb), lambda i: (0, i + off)),
                  full(ew1), full(eb1), full(we), full(cb1), full(emw2),
                  full(emb2), full(be), full(nb1), full(nw2), full(nb2)],
        out_specs=[pl.BlockSpec((16, eb), lambda i: (0, i)),
                   pl.BlockSpec((eb, 128), lambda i: (i, 0)),
                   pl.BlockSpec((16, 1), lambda i: (0, 0))],
        out_shape=[jax.ShapeDtypeStruct((16, _EH), jnp.float32),
                   jax.ShapeDtypeStruct((_EH, 128), jnp.float32),
                   jax.ShapeDtypeStruct((16, 1), jnp.float32)],
    )(gr, ea_t, ew1, eb1, we, cb1, emw2, emb2, be, nb1, nw2, nb2)


@functools.partial(
    pl.kernel,
    out_type=jax.ShapeDtypeStruct((2, _N, 128), jnp.float32),
    mesh=_mesh,
    scratch_types=[
        pltpu.VMEM((_ITERS, _CHUNK), jnp.int32),
        pltpu.VMEM((_CHUNK, 128), jnp.float32),
        pltpu.VMEM((_CHUNK, 128), jnp.float32),
        pltpu.VMEM((_CHUNK, 128), jnp.float32),
        pltpu.VMEM_SHARED((_N, 128), jnp.float32),
        pltpu.SemaphoreType.DMA,
        pltpu.SemaphoreType.DMA,
        pltpu.SemaphoreType.DMA,
        pltpu.SemaphoreType.DMA,
        pltpu.SemaphoreType.DMA,
        pltpu.SemaphoreType.DMA,
    ],
)
def _sc_scatter(h_hbm, col3_hbm, zeros_hbm, out_hbm, colv, hv0, hv1, hv2,
                acc, sh0, sh1, sh2, ss0, ss1, ss2):
    cid = lax.axis_index("c")
    sid = lax.axis_index("s")
    wid = sid * _NC + cid
    hvs = (hv0, hv1, hv2)
    shs = (sh0, sh1, sh2)
    sss = (ss0, ss1, ss2)

    @pl.when(sid == 0)
    def _():
        pltpu.sync_copy(zeros_hbm, acc)

    pltpu.sync_copy(col3_hbm.at[wid], colv)
    plsc.subcore_barrier()

    base0 = wid * _EPW

    def start(i, s):
        pltpu.async_copy(h_hbm.at[pl.ds(base0 + i * _CHUNK, _CHUNK)],
                         hvs[s], shs[s])

    def finish(i, s):
        pltpu.make_async_copy(
            h_hbm.at[pl.ds(base0 + i * _CHUNK, _CHUNK)], hvs[s],
            shs[s]).wait()
        pltpu.async_copy(hvs[s], acc.at[colv.at[i]], sss[s], add=True)

    def wait_sc(i, s):
        pltpu.make_async_copy(hvs[s], acc.at[colv.at[i]], sss[s]).wait()

    start(0, 0)

    def body(j, carry):
        i = 3 * j

        @pl.when(j > 0)
        def _():
            wait_sc(i - 2, 1)

        start(i + 1, 1)
        finish(i, 0)

        @pl.when(j > 0)
        def _():
            wait_sc(i - 1, 2)

        start(i + 2, 2)
        finish(i + 1, 1)

        wait_sc(i, 0)
        start(i + 3, 0)
        finish(i + 2, 2)
        return carry

    lax.fori_loop(0, (_ITERS - 2) // 3, body, 0)
    last = _ITERS - 2
    wait_sc(last - 2, 1)
    start(last + 1, 1)
    finish(last, 0)
    finish(last + 1, 1)
    wait_sc(last - 1, 2)
    wait_sc(last, 0)
    wait_sc(last + 1, 1)
    plsc.subcore_barrier()

    @pl.when(sid == 0)
    def _():
        pltpu.sync_copy(acc, out_hbm.at[cid])


def _final_body(xe_ref, pa_ref, pb_ref, easum_ref, cx_ref, ca_ref, nb1_ref,
                nw2_ref, nb2_ref, gx_ref, ge_ref, gb1_ref, gw2_ref, gb2_ref,
                sw1_ref, sb1_ref, sw2_ref, sb2_ref,
                xo_ref, sh_ref, u_ref, nacc_ref):
    agg = (pa_ref[0][:, :64] + pa_ref[1][:, :64]
           + pb_ref[0][:, :64] + pb_ref[1][:, :64])
    h2 = jnp.maximum(
        (xe_ref[...] @ cx_ref[...] + agg @ ca_ref[...] + nb1_ref[...]) * _BN,
        0.0)
    xo = h2 @ nw2_ref[...] + nb2_ref[...]
    xo_ref[...] = xo
    hs = jnp.maximum(xo @ sw1_ref[...] + sb1_ref[...], 0.0)
    sh_ref[...] = hs @ sw2_ref[...] + sb2_ref[...]

    i = pl.program_id(0)

    @pl.when(i == 0)
    def _():
        nacc_ref[...] = jnp.zeros_like(nacc_ref)

    nacc_ref[...] += jnp.sum(xo, axis=0, keepdims=True)

    @pl.when(i == pl.num_programs(0) - 1)
    def _():
        node_mean = nacc_ref[...] * (1.0 / _N)
        em_t = easum_ref[...] * (1.0 / _E)
        emg = lax.dot_general(em_t, ge_ref[...], (((0,), (0,)), ((), ())))
        gh = jnp.maximum(
            (node_mean @ gx_ref[...] + emg + gb1_ref[...]) * _BN, 0.0)
        u_ref[...] = gh @ gw2_ref[...] + gb2_ref[...]


def _final_stage(xe, pa, pb, easum, cx, ca, nb1, nw2, nb2, gx, ge, gb1, gw2,
                 gb2, sw1, sb1, sw2, sb2):
    nb = 1000
    grid = _N // nb
    full = lambda a: pl.BlockSpec(a.shape, lambda i: (0,) * a.ndim)
    return pl.pallas_call(
        _final_body,
        grid=(grid,),
        in_specs=[pl.BlockSpec((nb, 64), lambda i: (i, 0)),
                  pl.BlockSpec((2, nb, 128), lambda i: (0, i, 0)),
                  pl.BlockSpec((2, nb, 128), lambda i: (0, i, 0)),
                  full(easum), full(cx), full(ca), full(nb1), full(nw2),
                  full(nb2), full(gx), full(ge), full(gb1), full(gw2),
                  full(gb2), full(sw1), full(sb1), full(sw2), full(sb2)],
        out_specs=[pl.BlockSpec((nb, 64), lambda i: (i, 0)),
                   pl.BlockSpec((nb, 1), lambda i: (i, 0)),
                   pl.BlockSpec((1, 64), lambda i: (0, 0))],
        out_shape=[jax.ShapeDtypeStruct((_N, 64), jnp.float32),
                   jax.ShapeDtypeStruct((_N, 1), jnp.float32),
                   jax.ShapeDtypeStruct((1, 64), jnp.float32)],
        scratch_shapes=[pltpu.VMEM((1, 64), jnp.float32)],
    )(xe, pa, pb, easum, cx, ca, nb1, nw2, nb2, gx, ge, gb1, gw2, gb2, sw1,
      sb1, sw2, sb2)


def _idx_stream(r, c):
    return jnp.stack([r.reshape(-1, _CHUNK), c.reshape(-1, _CHUNK)],
                     axis=1).reshape(-1)


def kernel(x, edge_index, edge_attr, params):
    p = params
    row = edge_index[0]
    col = edge_index[1]
    r1 = lambda b: b.reshape(1, -1)

    a_r = p['em_W1'][:64]
    a_c = p['em_W1'][64:128]
    a_e = p['em_W1'][128:]
    b_r = p['nm1_W1'][:64]
    b_e = p['nm1_W1'][64:]
    wtr = jnp.concatenate([a_r, b_r], axis=1)
    w_e = p['ee_W2'] @ a_e
    cb1 = r1(p['ee_b2'] @ a_e + p['em_b1'])
    wtc = jnp.concatenate([a_c, jnp.zeros((64, 64), jnp.float32)], axis=1)

    idx_a = _idx_stream(row[:_EH], col[:_EH])
    idx_b = _idx_stream(row[_EH:], col[_EH:])
    col3_a = col[:_EH].reshape(_NW, _ITERS, _CHUNK)
    col3_b = col[_EH:].reshape(_NW, _ITERS, _CHUNK)
    zeros = jnp.zeros((_N, 128), jnp.float32)

    xe, tr, tcp = _encode(x, p['ne_W1'], r1(p['ne_b1']), p['ne_W2'],
                          r1(p['ne_b2']), wtr, wtc)
    g_a = _sc_gather(tr, tcp, idx_a)
    g_b = _sc_gather(tr, tcp, idx_b)
    ea_t = edge_attr.T
    rc = lambda b: b.reshape(-1, 1)
    edge_args = (p['ee_W1'], rc(p['ee_b1']), w_e, cb1,
                 p['em_W2'], rc(p['em_b2']), b_e, r1(p['nm1_b1']),
                 p['nm1_W2'], r1(p['nm1_b2']))
    ea2_a, h_a, easum_a = _edge_stage(g_a, ea_t, 0, *edge_args)
    ea2_b, h_b, easum_b = _edge_stage(g_b, ea_t, 1, *edge_args)
    parts_a = _sc_scatter(h_a, col3_a, zeros)
    parts_b = _sc_scatter(h_b, col3_b, zeros)
    ea2 = jnp.concatenate([ea2_a, ea2_b], axis=1).T
    easum = easum_a + easum_b
    xo, shifts, u = _final_stage(
        xe, parts_a, parts_b, easum, p['nm2_W1'][:64], p['nm2_W1'][64:],
        r1(p['nm2_b1']), p['nm2_W2'], r1(p['nm2_b2']),
        p['gm_W1'][:64], p['gm_W1'][64:], r1(p['gm_b1']), p['gm_W2'],
        r1(p['gm_b2']), p['sp_W1'], r1(p['sp_b1']), p['sp_W2'],
        r1(p['sp_b2']))
    return (shifts, (xo, ea2, u))
